# baseline (speedup 1.0000x reference)
import functools

import jax
import jax.numpy as jnp
from jax import lax
from jax.experimental import pallas as pl
from jax.experimental.pallas import tpu as pltpu
from jax.experimental.pallas import tpu_sc as plsc

_B, _C, _HW = 64, 768, 576
_NC, _NS = 2, 16
_NW = _NC * _NS
_BPW = _B // _NW
_RC = 48
_NCHUNK = _HW // _RC
_NV = _C // 16


def _sc_pool_body(x_hbm, out_hbm, buf0, buf1, acc_v, sem0, sem1):
    cid = lax.axis_index("c")
    sid = lax.axis_index("s")
    w = sid * _NC + cid
    bufs = (buf0, buf1)
    sems = (sem0, sem1)

    for bi in range(_BPW):
        b = w * _BPW + bi
        pltpu.make_async_copy(x_hbm.at[b, pl.ds(0, _RC), :], bufs[0], sems[0]).start()
        carry = tuple(jnp.zeros((16,), jnp.float32) for _ in range(_NV))
        for ci in range(_NCHUNK):
            cur = ci % 2
            pltpu.make_async_copy(
                x_hbm.at[b, pl.ds(ci * _RC, _RC), :], bufs[cur], sems[cur]
            ).wait()
            if ci + 1 < _NCHUNK:
                nxt = (ci + 1) % 2
                pltpu.make_async_copy(
                    x_hbm.at[b, pl.ds((ci + 1) * _RC, _RC), :], bufs[nxt], sems[nxt]
                ).start()
            buf = bufs[cur]

            def row_body(r, c, buf=buf):
                return tuple(
                    c[k] + buf[r, pl.ds(16 * k, 16)] for k in range(_NV)
                )

            carry = lax.fori_loop(0, _RC, row_body, carry)
        for k in range(_NV):
            acc_v[pl.ds(16 * k, 16)] = carry[k]
        pltpu.sync_copy(acc_v, out_hbm.at[b])


@functools.partial(jax.jit, static_argnames=())
def _sc_pool(xt):
    mesh = plsc.VectorSubcoreMesh(core_axis_name="c", subcore_axis_name="s")
    return pl.kernel(
        _sc_pool_body,
        out_type=jax.ShapeDtypeStruct((_B, _C), jnp.float32),
        mesh=mesh,
        scratch_types=[
            pltpu.VMEM((_RC, _C), jnp.float32),
            pltpu.VMEM((_RC, _C), jnp.float32),
            pltpu.VMEM((_C,), jnp.float32),
            pltpu.SemaphoreType.DMA,
            pltpu.SemaphoreType.DMA,
        ],
    )(xt)


def _mlp_body(s_ref, w1_ref, b1_ref, w2_ref, b2_ref, out_ref):
    pooled = s_ref[...] * (1.0 / _HW)
    h = pooled @ w1_ref[...] + b1_ref[...]
    h = 0.5 * h * (1.0 + jax.lax.erf(h * (2.0 ** -0.5)))
    logits = h @ w2_ref[...] + b2_ref[...]
    m = jnp.max(logits, axis=-1, keepdims=True)
    e = jnp.exp(logits - m)
    out_ref[...] = e / jnp.sum(e, axis=-1, keepdims=True)


def _mlp(sums, W1, b1, W2, b2):
    E = W2.shape[1]
    return pl.pallas_call(
        _mlp_body,
        out_shape=jax.ShapeDtypeStruct((_B, E), jnp.float32),
    )(sums, W1, b1, W2, b2)


def kernel(x, W1, b1, W2, b2):
    B, C, H, W = x.shape
    hw = H * W
    xt = jnp.transpose(x, (0, 2, 3, 1)).reshape(B, hw, C)
    sums = _sc_pool(xt)
    return _mlp(sums, W1, b1, W2, b2)


# speedup vs baseline: 1.5036x; 1.5036x over previous
import jax
import jax.numpy as jnp
from jax import lax
from jax.experimental import pallas as pl
from jax.experimental.pallas import tpu as pltpu
from jax.experimental.pallas import tpu_sc as plsc

_B, _C, _HW = 64, 768, 576
_NC, _NS = 2, 16
_NW = _NC * _NS
_B_SC = 16
_B_TC = _B - _B_SC
_NHALF = _NW // _B_SC
_ROWS = _HW // _NHALF
_RC = 48
_NCHUNK = _ROWS // _RC
_NV = _C // 16
_BBLK = 4


def _sc_pool_body(x_hbm, out_hbm, buf0, buf1, acc_v, sem0, sem1):
    cid = lax.axis_index("c")
    sid = lax.axis_index("s")
    w = sid * _NC + cid
    b_local = w // _NHALF
    half = w % _NHALF
    b = _B_TC + b_local
    row0 = half * _ROWS
    bufs = (buf0, buf1)
    sems = (sem0, sem1)

    pltpu.make_async_copy(
        x_hbm.at[b, pl.ds(row0, _RC), :], bufs[0], sems[0]
    ).start()
    carry = tuple(jnp.zeros((16,), jnp.float32) for _ in range(_NV))
    for ci in range(_NCHUNK):
        cur = ci % 2
        pltpu.make_async_copy(
            x_hbm.at[b, pl.ds(row0 + ci * _RC, _RC), :], bufs[cur], sems[cur]
        ).wait()
        if ci + 1 < _NCHUNK:
            nxt = (ci + 1) % 2
            pltpu.make_async_copy(
                x_hbm.at[b, pl.ds(row0 + (ci + 1) * _RC, _RC), :],
                bufs[nxt], sems[nxt],
            ).start()
        buf = bufs[cur]

        def row_body(r, c, buf=buf):
            return tuple(c[k] + buf[r, pl.ds(16 * k, 16)] for k in range(_NV))

        carry = lax.fori_loop(0, _RC, row_body, carry)
    for k in range(_NV):
        acc_v[pl.ds(16 * k, 16)] = carry[k]
    pltpu.sync_copy(acc_v, out_hbm.at[half, b_local])


def _sc_pool(xt):
    mesh = plsc.VectorSubcoreMesh(core_axis_name="c", subcore_axis_name="s")
    return pl.kernel(
        _sc_pool_body,
        out_type=jax.ShapeDtypeStruct((_NHALF, _B_SC, _C), jnp.float32),
        mesh=mesh,
        scratch_types=[
            pltpu.VMEM((_RC, _C), jnp.float32),
            pltpu.VMEM((_RC, _C), jnp.float32),
            pltpu.VMEM((_C,), jnp.float32),
            pltpu.SemaphoreType.DMA,
            pltpu.SemaphoreType.DMA,
        ],
    )(xt)


def _mlp(pooled, w1_ref, b1_ref, w2_ref, b2_ref):
    h = pooled @ w1_ref[...] + b1_ref[...]
    h = 0.5 * h * (1.0 + jax.lax.erf(h * (2.0 ** -0.5)))
    logits = h @ w2_ref[...] + b2_ref[...]
    m = jnp.max(logits, axis=-1, keepdims=True)
    e = jnp.exp(logits - m)
    return e / jnp.sum(e, axis=-1, keepdims=True)


def _tc_body(x_ref, w1_ref, b1_ref, w2_ref, b2_ref, out_ref):
    hw = x_ref.shape[1]
    pooled = jnp.sum(x_ref[...], axis=1) * (1.0 / hw)
    out_ref[0, :, :] = _mlp(pooled, w1_ref, b1_ref, w2_ref, b2_ref)


def _tc_pool_mlp(xt, W1, b1, W2, b2):
    E = W2.shape[1]
    out = pl.pallas_call(
        _tc_body,
        grid=(_B_TC // _BBLK,),
        in_specs=[
            pl.BlockSpec((_BBLK, _HW, _C), lambda i: (i, 0, 0)),
            pl.BlockSpec((_C, W1.shape[1]), lambda i: (0, 0)),
            pl.BlockSpec((W1.shape[1],), lambda i: (0,)),
            pl.BlockSpec((W1.shape[1], E), lambda i: (0, 0)),
            pl.BlockSpec((E,), lambda i: (0,)),
        ],
        out_specs=pl.BlockSpec((1, _BBLK, E), lambda i: (i, 0, 0)),
        out_shape=jax.ShapeDtypeStruct((_B_TC // _BBLK, _BBLK, E), jnp.float32),
    )(xt, W1, b1, W2, b2)
    return out.reshape(_B_TC, E)


def _sc_mlp_body(s_ref, w1_ref, b1_ref, w2_ref, b2_ref, out_ref):
    pooled = (s_ref[0, :, :] + s_ref[1, :, :]) * (1.0 / _HW)
    out_ref[...] = _mlp(pooled, w1_ref, b1_ref, w2_ref, b2_ref)


def _sc_mlp(sc_sums, W1, b1, W2, b2):
    E = W2.shape[1]
    return pl.pallas_call(
        _sc_mlp_body,
        out_shape=jax.ShapeDtypeStruct((_B_SC, E), jnp.float32),
    )(sc_sums, W1, b1, W2, b2)


def kernel(x, W1, b1, W2, b2):
    B, C, H, W = x.shape
    hw = H * W
    xt = jnp.transpose(x, (0, 2, 3, 1)).reshape(B, hw, C)
    sc_sums = _sc_pool(xt)
    tc_out = _tc_pool_mlp(xt, W1, b1, W2, b2)
    sc_out = _sc_mlp(sc_sums, W1, b1, W2, b2)
    return jnp.concatenate([tc_out, sc_out], axis=0)


# speedup vs baseline: 2.1525x; 1.4315x over previous
import jax
import jax.numpy as jnp
import numpy as np
from jax.experimental import pallas as pl

_B, _C, _HW = 64, 768, 576
_BBLK = 2
_NSTEP = _B // (2 * _BBLK)


def _router_body(xa_ref, xb_ref, w1_ref, b1_ref, w2_ref, b2_ref, out_ref):
    pooled_a = jnp.sum(xa_ref[...], axis=1)
    pooled_b = jnp.sum(xb_ref[...], axis=1)
    pooled = jnp.concatenate([pooled_a, pooled_b], axis=0) * (1.0 / _HW)
    h = pooled @ w1_ref[...] + b1_ref[...]
    h = 0.5 * h * (1.0 + jax.lax.erf(h * (2.0 ** -0.5)))
    logits = h @ w2_ref[...] + b2_ref[...]
    m = jnp.max(logits, axis=-1, keepdims=True)
    e = jnp.exp(logits - m)
    out_ref[0, :, :] = e / jnp.sum(e, axis=-1, keepdims=True)


def kernel(x, W1, b1, W2, b2):
    B, C, H, W = x.shape
    hw = H * W
    E = W2.shape[1]
    xt = jnp.transpose(x, (0, 2, 3, 1)).reshape(B, hw, C)
    out = pl.pallas_call(
        _router_body,
        grid=(_NSTEP,),
        in_specs=[
            pl.BlockSpec((_BBLK, hw, C), lambda i: (i, 0, 0)),
            pl.BlockSpec((_BBLK, hw, C), lambda i: (i + _NSTEP, 0, 0)),
            pl.BlockSpec((C, W1.shape[1]), lambda i: (0, 0)),
            pl.BlockSpec((W1.shape[1],), lambda i: (0,)),
            pl.BlockSpec((W1.shape[1], E), lambda i: (0, 0)),
            pl.BlockSpec((E,), lambda i: (0,)),
        ],
        out_specs=pl.BlockSpec((1, 2 * _BBLK, E), lambda i: (i, 0, 0)),
        out_shape=jax.ShapeDtypeStruct((_NSTEP, 2 * _BBLK, E), jnp.float32),
    )(xt, xt, W1, b1, W2, b2)
    out = out.reshape(B, E)
    order = np.concatenate(
        [np.stack([np.arange(_BBLK * i, _BBLK * (i + 1)),
                   B // 2 + np.arange(_BBLK * i, _BBLK * (i + 1))]).reshape(-1)
         for i in range(_NSTEP)]
    )
    inv = np.argsort(order)
    return out[inv]
